# Initial kernel scaffold; baseline (speedup 1.0000x reference)
import jax
import jax.numpy as jnp
from jax.experimental import pallas as pl


def kernel(input, edge_index, edge_weight, W, b):
    raise NotImplementedError("write your pallas kernel here")


# speedup vs baseline: 1.0545x; 1.0545x over previous
import functools

import jax
import jax.numpy as jnp
from jax import lax
from jax.experimental import pallas as pl
from jax.experimental.pallas import tpu as pltpu
from jax.experimental.pallas import tpu_sc as plsc

N_NODES = 10000
D = 128

NC = 2
NS = 16
NW = NC * NS
CHUNK = 128
ROWS_PER_TILE = N_NODES // NW


def _matmul_body(x_ref, w_ref, o_ref):
    o_ref[...] = jnp.dot(x_ref[...], w_ref[...],
                         preferred_element_type=jnp.float32)


def _combine_body(p_ref, b_ref, o_ref):
    o_ref[...] = p_ref[0] + p_ref[1] + b_ref[...]


def _sc_body(n_chunks,
             src_hbm, dst_hbm, w_hbm, sup_hbm, out_hbm,
             src_v, dst_v, w_v, rows_v, acc_sh, sem):
    c = lax.axis_index("c")
    s = lax.axis_index("s")
    wid = s * NC + c

    pltpu.sync_copy(src_hbm.at[wid], src_v)
    pltpu.sync_copy(dst_hbm.at[wid], dst_v)
    pltpu.sync_copy(w_hbm.at[wid], w_v)

    zero = jnp.zeros((16,), jnp.float32)

    @pl.loop(0, 80)
    def _zero_rows(r):
        for a in range(8):
            rows_v[r, pl.ds(a * 16, 16)] = zero

    for k in range(8):
        blk = s * 8 + k

        @pl.when(blk < 125)
        def _():
            r0 = pl.multiple_of(blk * 80, 80)
            pltpu.sync_copy(rows_v.at[pl.ds(0, 80)],
                            acc_sh.at[pl.ds(r0, 80)])
    plsc.subcore_barrier()

    iota = lax.iota(jnp.int32, 16)

    @pl.loop(0, n_chunks)
    def _edge_chunk(j):
        pltpu.async_copy(sup_hbm.at[src_v.at[j]], rows_v, sem).wait()
        for a in range(8):
            wsub = w_v[j, pl.ds(a * 16, 16)]
            ridx = iota + a * 16

            @pl.loop(0, 16)
            def _cols(cc):
                for u in range(8):
                    cidx = jnp.full((16,), cc * 8 + u, jnp.int32)
                    vals = plsc.load_gather(rows_v, [ridx, cidx])
                    plsc.store_scatter(rows_v, [ridx, cidx], vals * wsub)
        pltpu.sync_copy(rows_v, acc_sh.at[dst_v.at[j]], add=True)

    plsc.subcore_barrier()

    for k in range(8):
        blk = s * 8 + k

        @pl.when(blk < 125)
        def _():
            r0 = pl.multiple_of(blk * 80, 80)
            pltpu.sync_copy(acc_sh.at[pl.ds(r0, 80)], rows_v.at[pl.ds(0, 80)])
            pltpu.sync_copy(rows_v.at[pl.ds(0, 80)],
                            out_hbm.at[c, pl.ds(r0, 80)])


def kernel(input, edge_index, edge_weight, W, b):
    n_edges = edge_weight.shape[0]
    per_w = -(-n_edges // (NW * CHUNK)) * CHUNK
    n_chunks = per_w // CHUNK
    ep = per_w * NW
    pad = ep - n_edges

    dst = edge_index[0].astype(jnp.int32)
    src = edge_index[1].astype(jnp.int32)
    w = edge_weight.astype(jnp.float32)
    if pad:
        zi = jnp.zeros((pad,), jnp.int32)
        dst = jnp.concatenate([dst, zi])
        src = jnp.concatenate([src, zi])
        w = jnp.concatenate([w, jnp.zeros((pad,), jnp.float32)])
    src3 = src.reshape(NW, n_chunks, CHUNK)
    dst3 = dst.reshape(NW, n_chunks, CHUNK)
    w3 = w.reshape(NW, n_chunks, CHUNK)

    n = input.shape[0]
    blk = 1000
    support = pl.pallas_call(
        _matmul_body,
        grid=(n // blk,),
        in_specs=[pl.BlockSpec((blk, D), lambda i: (i, 0)),
                  pl.BlockSpec((D, D), lambda i: (0, 0))],
        out_specs=pl.BlockSpec((blk, D), lambda i: (i, 0)),
        out_shape=jax.ShapeDtypeStruct((n, D), jnp.float32),
    )(input, W)

    mesh = plsc.VectorSubcoreMesh(core_axis_name="c", subcore_axis_name="s")
    partial = pl.kernel(
        functools.partial(_sc_body, n_chunks),
        out_type=jax.ShapeDtypeStruct((NC, N_NODES, D), jnp.float32),
        mesh=mesh,
        compiler_params=pltpu.CompilerParams(needs_layout_passes=False),
        scratch_types=[
            pltpu.VMEM((n_chunks, CHUNK), jnp.int32),
            pltpu.VMEM((n_chunks, CHUNK), jnp.int32),
            pltpu.VMEM((n_chunks, CHUNK), jnp.float32),
            pltpu.VMEM((CHUNK, D), jnp.float32),
            pltpu.VMEM_SHARED((N_NODES, D), jnp.float32),
            pltpu.SemaphoreType.DMA,
        ],
    )(src3, dst3, w3, support)

    out = pl.pallas_call(
        _combine_body,
        grid=(n // blk,),
        in_specs=[pl.BlockSpec((NC, blk, D), lambda i: (0, i, 0)),
                  pl.BlockSpec((D,), lambda i: (0,))],
        out_specs=pl.BlockSpec((blk, D), lambda i: (i, 0)),
        out_shape=jax.ShapeDtypeStruct((n, D), jnp.float32),
    )(partial, b)
    return out
